# baseline (speedup 1.0000x reference)
import functools

import jax
import jax.numpy as jnp
from jax import lax
from jax.experimental import pallas as pl
from jax.experimental.pallas import tpu as pltpu
from jax.experimental.pallas import tpu_sc as plsc

N = 10000
D = 128
NR = 200
SQC = 0.1
NC = 2
NS = 16
NW = NC * NS
K = 80
CH = 126
KR = 96
CHR = 105
E_PAD = NW * CH * K
N_PAD = 10240
ZT = N_PAD // NS
REP = 32
RB = N_PAD // 8


def _sc_scatter_pass(table, idx, zeros, zeros1d, with_deg, k, ch, nb,
                     dtype=jnp.float32, init_full=False):
    mesh = plsc.VectorSubcoreMesh(core_axis_name="c", subcore_axis_name="s")
    out_type = [jax.ShapeDtypeStruct((NC, N_PAD, D), dtype)]
    if with_deg:
        out_type.append(jax.ShapeDtypeStruct((NW, N_PAD), jnp.float32))
    scratch = [pltpu.VMEM_SHARED((N_PAD, D), dtype)]
    for _ in range(nb):
        scratch += [pltpu.VMEM((2, k), jnp.int32),
                    pltpu.VMEM((k,), jnp.int32),
                    pltpu.VMEM((k, D), dtype)]
    if with_deg:
        scratch.append(pltpu.VMEM((N_PAD,), jnp.float32))
    scratch += [pltpu.SemaphoreType.DMA] * (3 * nb)

    @functools.partial(
        pl.kernel,
        out_type=tuple(out_type),
        mesh=mesh,
        compiler_params=pltpu.CompilerParams(needs_layout_passes=False),
        scratch_types=scratch,
    )
    def body(table_ref, idx_ref, zeros_ref, zeros1d_ref, *refs):
        n_out = 2 if with_deg else 1
        out_ref = refs[0]
        deg_out = refs[1] if with_deg else None
        acc = refs[n_out]
        bufs = [tuple(refs[n_out + 1 + 3 * b: n_out + 4 + 3 * b])
                for b in range(nb)]
        deg_t = refs[n_out + 1 + 3 * nb] if with_deg else None
        sems = refs[-3 * nb:]
        sets = [bufs[b] + tuple(sems[3 * b: 3 * b + 3]) for b in range(nb)]
        c = lax.axis_index("c")
        s = lax.axis_index("s")
        wid = s * NC + c
        if init_full:
            pltpu.sync_copy(zeros_ref.at[c, pl.ds(s * ZT, ZT)],
                            acc.at[pl.ds(s * ZT, ZT)])
        else:
            pltpu.sync_copy(zeros_ref, acc.at[pl.ds(s * ZT, ZT)])
        if with_deg:
            pltpu.sync_copy(zeros1d_ref, deg_t)
        plsc.subcore_barrier()

        ones16 = jnp.ones((16,), jnp.float32)

        def histo(ib):
            if with_deg:
                def h(j, carry):
                    dvec = ib[1, pl.ds(j * 16, 16)]
                    plsc.addupdate_scatter(deg_t, [dvec], ones16)
                    return carry
                lax.fori_loop(0, k // 16, h, 0)

        def cpidx(ib, sx):
            def cp(j, carry):
                sx[pl.ds(j * 16, 16)] = ib[1, pl.ds(j * 16, 16)]
                return carry
            lax.fori_loop(0, k // 16, cp, 0)

        def start(i, st):
            ib, sx, rows, sem_i, sem_g, sem_s = st
            pltpu.make_async_copy(idx_ref.at[wid, i], ib, sem_i).wait()

            @pl.when(i >= nb)
            def _():
                pltpu.make_async_copy(rows, acc.at[sx], sem_s).wait()

            cpidx(ib, sx)
            pltpu.async_copy(table_ref.at[ib.at[0]], rows, sem_g)

        def finish(i, st):
            ib, sx, rows, sem_i, sem_g, sem_s = st
            pltpu.make_async_copy(table_ref.at[ib.at[0]], rows, sem_g).wait()
            pltpu.async_copy(rows, acc.at[sx], sem_s, add=True)

            @pl.when(i + nb < ch)
            def _():
                pltpu.async_copy(idx_ref.at[wid, i + nb], ib, sem_i)

        for kk in range(nb):
            ib, _, _, sem_i = sets[kk][:4]
            pltpu.async_copy(idx_ref.at[wid, kk], ib, sem_i)

        def step(i, carry):
            for kk in range(nb):
                start(i + kk, sets[kk])
            for kk in range(nb):
                histo(sets[kk][0])
            for kk in range(nb):
                finish(i + kk, sets[kk])
            return carry

        nsteps = ch // nb
        lax.fori_loop(0, nsteps, lambda j, x: step(nb * j, x), 0)
        for kk in range(ch - nb * nsteps):
            i = jnp.int32(nb * nsteps + kk)
            start(i, sets[kk])
            histo(sets[kk][0])
            finish(i, sets[kk])
        for st in sets:
            _, sx, rows, _, _, sem_s = st
            pltpu.make_async_copy(rows, acc.at[sx], sem_s).wait()
        plsc.subcore_barrier()
        pltpu.sync_copy(acc.at[pl.ds(s * ZT, ZT)],
                        out_ref.at[c, pl.ds(s * ZT, ZT)])
        if with_deg:
            pltpu.sync_copy(deg_t, deg_out.at[wid])

    return body(table, idx, zeros, zeros1d)


def _sc_rel_agg_pass(trel, idx_rel, t0, idx_agg, zeros, zeros1d, k, ch, nb):
    mesh = plsc.VectorSubcoreMesh(core_axis_name="c", subcore_axis_name="s")
    out_type = (jax.ShapeDtypeStruct((NC, N_PAD, D), jnp.float32),
                jax.ShapeDtypeStruct((NC, N_PAD, D), jnp.float32),
                jax.ShapeDtypeStruct((NW, N_PAD), jnp.float32))
    scratch = [pltpu.VMEM_SHARED((N_PAD, D), jnp.float32)]
    for _ in range(nb):
        scratch += [pltpu.VMEM((2, k), jnp.int32),
                    pltpu.VMEM((k,), jnp.int32),
                    pltpu.VMEM((k, D), jnp.float32)]
    scratch.append(pltpu.VMEM((N_PAD,), jnp.float32))
    scratch += [pltpu.SemaphoreType.DMA] * (3 * nb)

    @functools.partial(
        pl.kernel,
        out_type=out_type,
        mesh=mesh,
        compiler_params=pltpu.CompilerParams(needs_layout_passes=False),
        scratch_types=scratch,
    )
    def body(trel_ref, idxr_ref, t0_ref, idxa_ref, zeros_ref, zeros1d_ref,
             relp_o, aggp_o, degp_o, *refs):
        acc = refs[0]
        bufs = [tuple(refs[1 + 3 * b: 4 + 3 * b]) for b in range(nb)]
        deg_t = refs[1 + 3 * nb]
        sems = refs[-3 * nb:]
        sets = [bufs[b] + tuple(sems[3 * b: 3 * b + 3]) for b in range(nb)]
        c = lax.axis_index("c")
        s = lax.axis_index("s")
        wid = s * NC + c
        pltpu.sync_copy(zeros_ref, acc.at[pl.ds(s * ZT, ZT)])
        pltpu.sync_copy(zeros1d_ref, deg_t)
        plsc.subcore_barrier()

        ones16 = jnp.ones((16,), jnp.float32)

        def cpidx(ib, sx):
            def cp(j, carry):
                sx[pl.ds(j * 16, 16)] = ib[1, pl.ds(j * 16, 16)]
                return carry
            lax.fori_loop(0, k // 16, cp, 0)

        def run_phase(tbl, idxr, do_histo):
            def histo(ib):
                if do_histo:
                    def h(j, carry):
                        dvec = ib[1, pl.ds(j * 16, 16)]
                        plsc.addupdate_scatter(deg_t, [dvec], ones16)
                        return carry
                    lax.fori_loop(0, k // 16, h, 0)

            def start(i, st):
                ib, sx, rows, sem_i, sem_g, sem_s = st
                pltpu.make_async_copy(idxr.at[wid, i], ib, sem_i).wait()

                @pl.when(i >= nb)
                def _():
                    pltpu.make_async_copy(rows, acc.at[sx], sem_s).wait()

                cpidx(ib, sx)
                pltpu.async_copy(tbl.at[ib.at[0]], rows, sem_g)

            def finish(i, st):
                ib, sx, rows, sem_i, sem_g, sem_s = st
                pltpu.make_async_copy(tbl.at[ib.at[0]], rows, sem_g).wait()
                pltpu.async_copy(rows, acc.at[sx], sem_s, add=True)

                @pl.when(i + nb < ch)
                def _():
                    pltpu.async_copy(idxr.at[wid, i + nb], ib, sem_i)

            for kk in range(nb):
                ib, _, _, sem_i = sets[kk][:4]
                pltpu.async_copy(idxr.at[wid, kk], ib, sem_i)

            def step(i, carry):
                for kk in range(nb):
                    start(i + kk, sets[kk])
                for kk in range(nb):
                    histo(sets[kk][0])
                for kk in range(nb):
                    finish(i + kk, sets[kk])
                return carry

            nsteps = ch // nb
            lax.fori_loop(0, nsteps, lambda j, x: step(nb * j, x), 0)
            for kk in range(ch - nb * nsteps):
                i = jnp.int32(nb * nsteps + kk)
                start(i, sets[kk])
                histo(sets[kk][0])
                finish(i, sets[kk])
            for st in sets:
                _, sx, rows, _, _, sem_s = st
                pltpu.make_async_copy(rows, acc.at[sx], sem_s).wait()

        run_phase(trel_ref, idxr_ref, True)
        plsc.subcore_barrier()
        pltpu.sync_copy(acc.at[pl.ds(s * ZT, ZT)],
                        relp_o.at[c, pl.ds(s * ZT, ZT)])
        pltpu.sync_copy(deg_t, degp_o.at[wid])
        plsc.subcore_barrier()
        run_phase(t0_ref, idxa_ref, False)
        plsc.subcore_barrier()
        pltpu.sync_copy(acc.at[pl.ds(s * ZT, ZT)],
                        aggp_o.at[c, pl.ds(s * ZT, ZT)])

    return body(trel, idx_rel, t0, idx_agg, zeros, zeros1d)


def _expmap0(u):
    n = jnp.maximum(jnp.sqrt(jnp.sum(u * u, axis=-1, keepdims=True)), 1e-10)
    return jnp.tanh(SQC * n) * u / (SQC * n)


def _logmap0(p):
    n = jnp.maximum(jnp.sqrt(jnp.sum(p * p, axis=-1, keepdims=True)), 1e-10)
    arg = jnp.clip(SQC * n, 0.0, 1.0 - 1e-5)
    atanh = 0.5 * (jnp.log1p(arg) - jnp.log1p(-arg))
    return atanh * p / (SQC * n)


def _layer_tangent(aggp_ref, degp_ref, t_ref, w_ref):
    agg = aggp_ref[0] + aggp_ref[1]
    deg = jnp.sum(degp_ref[...], axis=1, keepdims=True)
    a = jnp.dot(agg, w_ref[...], preferred_element_type=jnp.float32)
    a = a / jnp.maximum(deg, 1.0) + t_ref[...]
    a = jnp.where(a >= 0, a, a * ((1.0 / 8.0 + 1.0 / 3.0) / 2.0))
    return _logmap0(_expmap0(a))


def _tc_b_body(aggp_ref, degp_ref, t_ref, w_ref, out_ref):
    out_ref[...] = _layer_tangent(aggp_ref, degp_ref, t_ref, w_ref)


def _tc_c_body(aggp_ref, degp_ref, t1_ref, t0_ref, w_ref, w1_ref,
               w2_ref, out_ref):
    t2 = _layer_tangent(aggp_ref, degp_ref, t1_ref, w_ref)
    t0 = t0_ref[...]
    g = jax.nn.sigmoid(
        jnp.dot(t2, w1_ref[...], preferred_element_type=jnp.float32)
        + jnp.dot(t0, w2_ref[...], preferred_element_type=jnp.float32))
    out_ref[...] = _expmap0(g * t2 + (1.0 - g) * t0)


def _row_spec(width=D):
    return pl.BlockSpec((RB, width), lambda i: (i, 0))


def _part_spec(width):
    return pl.BlockSpec((NC, RB, width), lambda i: (0, i, 0))


def _deg_spec():
    return pl.BlockSpec((RB, NW), lambda i: (i, 0))


def _w_spec():
    return pl.BlockSpec((D, D), lambda i: (0, 0))


_GRID = (N_PAD // RB,)
_OUT_T = jax.ShapeDtypeStruct((N_PAD, D), jnp.float32)


def kernel(edge_index, edge_type, dynamic_emb, emb_rel, W_layers, w1, w2):
    src = edge_index[0].astype(jnp.int32)
    dst = edge_index[1].astype(jnp.int32)
    et = edge_type.astype(jnp.int32)
    e = src.shape[0]
    pidx = jnp.arange(E_PAD - e, dtype=jnp.int32)
    src_f = jnp.concatenate([src, pidx % N])
    dst_f = jnp.concatenate([dst, N + pidx % (N_PAD - N)])
    et_f = (jnp.concatenate([et, pidx % NR])
            + NR * (jnp.arange(E_PAD, dtype=jnp.int32) % REP))
    idx_agg = jnp.stack([src_f.reshape(NW, CH, K),
                         dst_f.reshape(NW, CH, K)], axis=2)
    idx_rel = jnp.stack([et_f.reshape(NW, CH, K),
                         dst_f.reshape(NW, CH, K)], axis=2)
    trel = jnp.tile(emb_rel, (REP, 1))

    zeros_d = jnp.zeros((ZT, D), jnp.float32)
    zeros1d = jnp.zeros((N_PAD,), jnp.float32)
    t0 = jnp.pad(dynamic_emb, ((0, N_PAD - N), (0, 0)))

    relp, agg0, degp = _sc_rel_agg_pass(trel, idx_rel, t0, idx_agg,
                                        zeros_d, zeros1d, K, CH, 3)
    degp = degp.T

    t1 = pl.pallas_call(
        _tc_b_body, grid=_GRID,
        in_specs=[_part_spec(D), _deg_spec(), _row_spec(), _w_spec()],
        out_specs=_row_spec(), out_shape=_OUT_T,
    )(agg0, degp, t0, W_layers[0])

    agg1, = _sc_scatter_pass(t1, idx_agg, relp, zeros1d, False, K, CH, 4,
                             init_full=True)

    out = pl.pallas_call(
        _tc_c_body, grid=_GRID,
        in_specs=[_part_spec(D), _deg_spec(), _row_spec(),
                  _row_spec(), _w_spec(), _w_spec(), _w_spec()],
        out_specs=_row_spec(), out_shape=_OUT_T,
    )(agg1, degp, t1, t0, W_layers[1], w1, w2)

    return out[:N]


# speedup vs baseline: 1.0346x; 1.0346x over previous
import functools

import jax
import jax.numpy as jnp
from jax import lax
from jax.experimental import pallas as pl
from jax.experimental.pallas import tpu as pltpu
from jax.experimental.pallas import tpu_sc as plsc

N = 10000
D = 128
NR = 200
SQC = 0.1
NC = 2
NS = 16
NW = NC * NS
K = 80
CH = 126
KR = 96
CHR = 105
E_PAD = NW * CH * K
N_PAD = 10240
ZT = N_PAD // NS
REP = 32
RB = N_PAD // 8


def _sc_scatter_pass(table, idx, zeros, zeros1d, with_deg, k, ch, nb,
                     dtype=jnp.float32, init_full=False):
    mesh = plsc.VectorSubcoreMesh(core_axis_name="c", subcore_axis_name="s")
    out_type = [jax.ShapeDtypeStruct((NC, N_PAD, D), dtype)]
    if with_deg:
        out_type.append(jax.ShapeDtypeStruct((NW, N_PAD), jnp.float32))
    scratch = [pltpu.VMEM_SHARED((N_PAD, D), dtype)]
    for _ in range(nb):
        scratch += [pltpu.VMEM((2, k), jnp.int32),
                    pltpu.VMEM((k,), jnp.int32),
                    pltpu.VMEM((k, D), dtype)]
    if with_deg:
        scratch.append(pltpu.VMEM((N_PAD,), jnp.float32))
    scratch += [pltpu.SemaphoreType.DMA] * (3 * nb)

    @functools.partial(
        pl.kernel,
        out_type=tuple(out_type),
        mesh=mesh,
        compiler_params=pltpu.CompilerParams(needs_layout_passes=False),
        scratch_types=scratch,
    )
    def body(table_ref, idx_ref, zeros_ref, zeros1d_ref, *refs):
        n_out = 2 if with_deg else 1
        out_ref = refs[0]
        deg_out = refs[1] if with_deg else None
        acc = refs[n_out]
        bufs = [tuple(refs[n_out + 1 + 3 * b: n_out + 4 + 3 * b])
                for b in range(nb)]
        deg_t = refs[n_out + 1 + 3 * nb] if with_deg else None
        sems = refs[-3 * nb:]
        sets = [bufs[b] + tuple(sems[3 * b: 3 * b + 3]) for b in range(nb)]
        c = lax.axis_index("c")
        s = lax.axis_index("s")
        wid = s * NC + c
        if init_full:
            pltpu.sync_copy(zeros_ref.at[c, pl.ds(s * ZT, ZT)],
                            acc.at[pl.ds(s * ZT, ZT)])
        else:
            pltpu.sync_copy(zeros_ref, acc.at[pl.ds(s * ZT, ZT)])
        if with_deg:
            pltpu.sync_copy(zeros1d_ref, deg_t)
        plsc.subcore_barrier()

        ones16 = jnp.ones((16,), jnp.float32)

        def histo(ib):
            if with_deg:
                def h(j, carry):
                    dvec = ib[1, pl.ds(j * 16, 16)]
                    plsc.addupdate_scatter(deg_t, [dvec], ones16)
                    return carry
                lax.fori_loop(0, k // 16, h, 0)

        def cpidx(ib, sx):
            def cp(j, carry):
                sx[pl.ds(j * 16, 16)] = ib[1, pl.ds(j * 16, 16)]
                return carry
            lax.fori_loop(0, k // 16, cp, 0)

        def start(i, st):
            ib, sx, rows, sem_i, sem_g, sem_s = st
            pltpu.make_async_copy(idx_ref.at[wid, i], ib, sem_i).wait()

            @pl.when(i >= nb)
            def _():
                pltpu.make_async_copy(rows, acc.at[sx], sem_s).wait()

            cpidx(ib, sx)
            pltpu.async_copy(table_ref.at[ib.at[0]], rows, sem_g)

        def finish(i, st):
            ib, sx, rows, sem_i, sem_g, sem_s = st
            pltpu.make_async_copy(table_ref.at[ib.at[0]], rows, sem_g).wait()
            pltpu.async_copy(rows, acc.at[sx], sem_s, add=True)

            @pl.when(i + nb < ch)
            def _():
                pltpu.async_copy(idx_ref.at[wid, i + nb], ib, sem_i)

        for kk in range(nb):
            ib, _, _, sem_i = sets[kk][:4]
            pltpu.async_copy(idx_ref.at[wid, kk], ib, sem_i)

        def step(i, carry):
            for kk in range(nb):
                start(i + kk, sets[kk])
            for kk in range(nb):
                histo(sets[kk][0])
            for kk in range(nb):
                finish(i + kk, sets[kk])
            return carry

        nsteps = ch // nb
        lax.fori_loop(0, nsteps, lambda j, x: step(nb * j, x), 0)
        for kk in range(ch - nb * nsteps):
            i = jnp.int32(nb * nsteps + kk)
            start(i, sets[kk])
            histo(sets[kk][0])
            finish(i, sets[kk])
        for st in sets:
            _, sx, rows, _, _, sem_s = st
            pltpu.make_async_copy(rows, acc.at[sx], sem_s).wait()
        plsc.subcore_barrier()
        pltpu.sync_copy(acc.at[pl.ds(s * ZT, ZT)],
                        out_ref.at[c, pl.ds(s * ZT, ZT)])
        if with_deg:
            pltpu.sync_copy(deg_t, deg_out.at[wid])

    return body(table, idx, zeros, zeros1d)


def _expmap0(u):
    n = jnp.maximum(jnp.sqrt(jnp.sum(u * u, axis=-1, keepdims=True)), 1e-10)
    return jnp.tanh(SQC * n) * u / (SQC * n)


def _logmap0(p):
    n = jnp.maximum(jnp.sqrt(jnp.sum(p * p, axis=-1, keepdims=True)), 1e-10)
    arg = jnp.clip(SQC * n, 0.0, 1.0 - 1e-5)
    atanh = 0.5 * (jnp.log1p(arg) - jnp.log1p(-arg))
    return atanh * p / (SQC * n)


def _layer_tangent(aggp_ref, degp_ref, t_ref, w_ref):
    agg = aggp_ref[0] + aggp_ref[1]
    deg = jnp.sum(degp_ref[...], axis=1, keepdims=True)
    a = jnp.dot(agg, w_ref[...], preferred_element_type=jnp.float32)
    a = a / jnp.maximum(deg, 1.0) + t_ref[...]
    a = jnp.where(a >= 0, a, a * ((1.0 / 8.0 + 1.0 / 3.0) / 2.0))
    return _logmap0(_expmap0(a))


def _tc_b_body(aggp_ref, degp_ref, t_ref, w_ref, out_ref):
    out_ref[...] = _layer_tangent(aggp_ref, degp_ref, t_ref, w_ref)


def _tc_c_body(aggp_ref, degp_ref, t1_ref, t0_ref, w_ref, w1_ref,
               w2_ref, out_ref):
    t2 = _layer_tangent(aggp_ref, degp_ref, t1_ref, w_ref)
    t0 = t0_ref[...]
    g = jax.nn.sigmoid(
        jnp.dot(t2, w1_ref[...], preferred_element_type=jnp.float32)
        + jnp.dot(t0, w2_ref[...], preferred_element_type=jnp.float32))
    out_ref[...] = _expmap0(g * t2 + (1.0 - g) * t0)


def _row_spec(width=D):
    return pl.BlockSpec((RB, width), lambda i: (i, 0))


def _part_spec(width):
    return pl.BlockSpec((NC, RB, width), lambda i: (0, i, 0))


def _deg_spec():
    return pl.BlockSpec((RB, NW), lambda i: (i, 0))


def _w_spec():
    return pl.BlockSpec((D, D), lambda i: (0, 0))


_GRID = (N_PAD // RB,)
_OUT_T = jax.ShapeDtypeStruct((N_PAD, D), jnp.float32)


def kernel(edge_index, edge_type, dynamic_emb, emb_rel, W_layers, w1, w2):
    src = edge_index[0].astype(jnp.int32)
    dst = edge_index[1].astype(jnp.int32)
    et = edge_type.astype(jnp.int32)
    e = src.shape[0]
    pidx = jnp.arange(E_PAD - e, dtype=jnp.int32)
    src_f = jnp.concatenate([src, pidx % N])
    dst_f = jnp.concatenate([dst, N + pidx % (N_PAD - N)])
    et_f = (jnp.concatenate([et, pidx % NR])
            + NR * (jnp.arange(E_PAD, dtype=jnp.int32) % REP))
    idx_agg = jnp.stack([src_f.reshape(NW, CH, K),
                         dst_f.reshape(NW, CH, K)], axis=2)
    idx_rel = jnp.stack([et_f.reshape(NW, CHR, KR),
                         dst_f.reshape(NW, CHR, KR)], axis=2)
    trel = jnp.tile(emb_rel, (REP, 1))

    zeros_d = jnp.zeros((ZT, D), jnp.float32)
    zeros1d = jnp.zeros((N_PAD,), jnp.float32)
    t0 = jnp.pad(dynamic_emb, ((0, N_PAD - N), (0, 0)))

    relp, degp = _sc_scatter_pass(trel, idx_rel, zeros_d, zeros1d, True,
                                  KR, CHR, 3)
    degp = degp.T
    agg0, = _sc_scatter_pass(t0, idx_agg, relp, zeros1d, False, K, CH, 4,
                             init_full=True)

    t1 = pl.pallas_call(
        _tc_b_body, grid=_GRID,
        in_specs=[_part_spec(D), _deg_spec(), _row_spec(), _w_spec()],
        out_specs=_row_spec(), out_shape=_OUT_T,
    )(agg0, degp, t0, W_layers[0])

    agg1, = _sc_scatter_pass(t1, idx_agg, relp, zeros1d, False, K, CH, 4,
                             init_full=True)

    out = pl.pallas_call(
        _tc_c_body, grid=_GRID,
        in_specs=[_part_spec(D), _deg_spec(), _row_spec(),
                  _row_spec(), _w_spec(), _w_spec(), _w_spec()],
        out_specs=_row_spec(), out_shape=_OUT_T,
    )(agg1, degp, t1, t0, W_layers[1], w1, w2)

    return out[:N]
